# baseline (speedup 1.0000x reference)
import functools

import jax
import jax.numpy as jnp
from jax import lax
from jax.experimental import pallas as pl
from jax.experimental.pallas import tpu as pltpu
from jax.experimental.pallas import tpu_sc as plsc

N_VOCAB = 49408
N_EMBD = 768
N_TOKENS = 77
BATCH = 256

NC = 2
NS = 16
L = 16
NW = NC * NS
SLOTS = BATCH * N_TOKENS
SLOTS_PER_W = SLOTS // NW
CHUNK = 56
NCHUNK = SLOTS_PER_W // CHUNK
NVEC = N_EMBD // L


def _make_kernel():
  mesh = plsc.VectorSubcoreMesh(core_axis_name="c", subcore_axis_name="s")

  @functools.partial(
      pl.kernel,
      mesh=mesh,
      out_type=jax.ShapeDtypeStruct((SLOTS, N_EMBD), jnp.float32),
      scratch_types=[
          pltpu.VMEM((SLOTS_PER_W,), jnp.int32),
          pltpu.VMEM((N_TOKENS, N_EMBD), jnp.float32),
          pltpu.VMEM((CHUNK, N_EMBD), jnp.float32),
          pltpu.SemaphoreType.DMA,
      ],
  )
  def emb_kernel(tok_hbm, table_hbm, pos_hbm, out_hbm, idx_v, pos_v, rows_v,
                 sem):
    wid = lax.axis_index("s") * NC + lax.axis_index("c")
    slot_base = wid * SLOTS_PER_W
    pltpu.sync_copy(pos_hbm, pos_v)
    pltpu.sync_copy(tok_hbm.at[pl.ds(slot_base, SLOTS_PER_W)], idx_v)

    def per_chunk(c, _):
      pltpu.async_copy(table_hbm.at[idx_v.at[pl.ds(c * CHUNK, CHUNK)]],
                       rows_v, sem).wait()

      def add_pos(i, _):
        p = lax.rem(c * CHUNK + i, N_TOKENS)
        for j in range(NVEC):
          sl = pl.ds(j * L, L)
          rows_v[i, sl] = rows_v[i, sl] + pos_v[p, sl]
        return 0

      lax.fori_loop(0, CHUNK, add_pos, 0)
      pltpu.sync_copy(rows_v, out_hbm.at[pl.ds(slot_base + c * CHUNK, CHUNK)])
      return 0

    lax.fori_loop(0, NCHUNK, per_chunk, 0)

  return emb_kernel


_EMB_KERNEL = _make_kernel()


def kernel(tokens, token_embedding, position_embedding):
  tok_flat = tokens.astype(jnp.int32).reshape(SLOTS)
  out = _EMB_KERNEL(tok_flat, token_embedding, position_embedding)
  return out.reshape(BATCH, N_TOKENS, N_EMBD)


# speedup vs baseline: 1.1080x; 1.1080x over previous
import functools

import jax
import jax.numpy as jnp
from jax import lax
from jax.experimental import pallas as pl
from jax.experimental.pallas import tpu as pltpu
from jax.experimental.pallas import tpu_sc as plsc

N_VOCAB = 49408
N_EMBD = 768
N_TOKENS = 77
BATCH = 256

NC = 2
NS = 16
L = 16
NW = NC * NS
SLOTS = BATCH * N_TOKENS
SLOTS_PER_W = SLOTS // NW
ROWS_PER_W = BATCH // NW
TPAD = 80
SUB = 24
NSUB = 4
GN = (SUB, SUB, SUB, 8)
WN = (SUB, SUB, SUB, N_TOKENS - 3 * SUB)
NVEC = N_EMBD // L


def _make_kernel():
  mesh = plsc.VectorSubcoreMesh(core_axis_name="c", subcore_axis_name="s")

  @functools.partial(
      pl.kernel,
      mesh=mesh,
      out_type=jax.ShapeDtypeStruct((BATCH, N_TOKENS, N_EMBD), jnp.float32),
      scratch_types=[
          pltpu.VMEM((ROWS_PER_W, TPAD), jnp.int32),
          pltpu.VMEM((N_TOKENS, N_EMBD), jnp.float32),
          pltpu.VMEM((SUB, N_EMBD), jnp.float32),
          pltpu.VMEM((SUB, N_EMBD), jnp.float32),
          pltpu.SemaphoreType.DMA,
          pltpu.SemaphoreType.DMA,
          pltpu.SemaphoreType.DMA,
          pltpu.SemaphoreType.DMA,
      ],
  )
  def emb_kernel(tok_hbm, table_hbm, pos_hbm, out_hbm, idx8_v, pos_v,
                 buf0, buf1, gsem0, gsem1, wsem0, wsem1):
    bufs = (buf0, buf1)
    gsems = (gsem0, gsem1)
    wsems = (wsem0, wsem1)
    wid = lax.axis_index("s") * NC + lax.axis_index("c")
    row_base = wid * ROWS_PER_W
    pltpu.sync_copy(pos_hbm, pos_v)
    pltpu.sync_copy(tok_hbm.at[pl.ds(row_base, ROWS_PER_W)], idx8_v)

    def gather_of(r, s, b):
      return pltpu.make_async_copy(
          table_hbm.at[idx8_v.at[r, pl.ds(s * SUB, GN[s])]],
          bufs[b].at[pl.ds(0, GN[s])], gsems[b])

    def write_of(r, s, b):
      return pltpu.make_async_copy(
          bufs[b].at[pl.ds(0, WN[s])],
          out_hbm.at[row_base + r, pl.ds(s * SUB, WN[s])], wsems[b])

    gather_of(0, 0, 0).start()

    def per_batch_row(r, _):
      for s in range(NSUB):
        b = s % 2
        ob = 1 - b
        gather_of(r, s, b).wait()

        def add_pos(i, _):
          t = s * SUB + i
          p = jnp.minimum(t, N_TOKENS - 1) if s == NSUB - 1 else t
          for j in range(NVEC):
            sl = pl.ds(j * L, L)
            bufs[b][i, sl] = bufs[b][i, sl] + pos_v[p, sl]
          return 0

        lax.fori_loop(0, GN[s], add_pos, 0)

        ps = (s - 1) % NSUB
        if s == 0:
          @pl.when(r >= 1)
          def _():
            write_of(r - 1, ps, ob).wait()
          gather_of(r, 1, ob).start()
        else:
          write_of(r, ps, ob).wait()
          if s == NSUB - 1:
            @pl.when(r <= ROWS_PER_W - 2)
            def _():
              gather_of(r + 1, 0, ob).start()
          else:
            gather_of(r, s + 1, ob).start()

        write_of(r, s, b).start()
      return 0

    lax.fori_loop(0, ROWS_PER_W, per_batch_row, 0)
    write_of(ROWS_PER_W - 1, NSUB - 1, (NSUB - 1) % 2).wait()

  return emb_kernel


_EMB_KERNEL = _make_kernel()


def kernel(tokens, token_embedding, position_embedding):
  tok_pad = jnp.pad(tokens.astype(jnp.int32), ((0, 0), (0, TPAD - N_TOKENS)))
  return _EMB_KERNEL(tok_pad, token_embedding, position_embedding)


# speedup vs baseline: 1.4975x; 1.3514x over previous
import functools

import jax
import jax.numpy as jnp
from jax import lax
from jax.experimental import pallas as pl
from jax.experimental.pallas import tpu as pltpu
from jax.experimental.pallas import tpu_sc as plsc

N_VOCAB = 49408
N_EMBD = 768
N_TOKENS = 77
BATCH = 256

NC = 2
NS = 16
L = 16
NW = NC * NS
ROWS_PER_W = BATCH // NW
TPAD = 80
NSUB = 4
T0 = (0, 24, 48, 72)
GN = (24, 24, 24, 8)
WN = (24, 24, 24, N_TOKENS - 72)
NVEC = N_EMBD // L


def _make_kernel():
  mesh = plsc.VectorSubcoreMesh(core_axis_name="c", subcore_axis_name="s")

  @functools.partial(
      pl.kernel,
      mesh=mesh,
      out_type=jax.ShapeDtypeStruct((BATCH, N_TOKENS, N_EMBD), jnp.float32),
      scratch_types=[
          pltpu.VMEM((ROWS_PER_W, TPAD), jnp.int32),
          pltpu.VMEM((N_TOKENS, N_EMBD), jnp.float32),
          pltpu.VMEM((GN[0], N_EMBD), jnp.float32),
          pltpu.VMEM((GN[1], N_EMBD), jnp.float32),
          pltpu.VMEM((GN[2], N_EMBD), jnp.float32),
          pltpu.VMEM((GN[3], N_EMBD), jnp.float32),
          pltpu.SemaphoreType.DMA,
          pltpu.SemaphoreType.DMA,
          pltpu.SemaphoreType.DMA,
          pltpu.SemaphoreType.DMA,
          pltpu.SemaphoreType.DMA,
          pltpu.SemaphoreType.DMA,
          pltpu.SemaphoreType.DMA,
          pltpu.SemaphoreType.DMA,
      ],
  )
  def emb_kernel(tok_hbm, table_hbm, pos_hbm, out_hbm, idx8_v, pos_v,
                 buf0, buf1, buf2, buf3, gsem0, gsem1, gsem2, gsem3,
                 wsem0, wsem1, wsem2, wsem3):
    bufs = (buf0, buf1, buf2, buf3)
    gsems = (gsem0, gsem1, gsem2, gsem3)
    wsems = (wsem0, wsem1, wsem2, wsem3)
    wid = lax.axis_index("s") * NC + lax.axis_index("c")
    row_base = wid * ROWS_PER_W

    def gather_of(r, s):
      return pltpu.make_async_copy(
          table_hbm.at[idx8_v.at[r, pl.ds(T0[s], GN[s])]],
          bufs[s].at[pl.ds(0, GN[s])], gsems[s])

    def write_of(r, s):
      return pltpu.make_async_copy(
          bufs[s].at[pl.ds(0, WN[s])],
          out_hbm.at[row_base + r, pl.ds(T0[s], WN[s])], wsems[s])

    pltpu.sync_copy(tok_hbm.at[pl.ds(row_base, ROWS_PER_W)], idx8_v)
    gather_of(0, 0).start()
    pltpu.sync_copy(pos_hbm, pos_v)

    def per_batch_row(r, _):
      for s in range(NSUB):
        ns = (s + 1) % NSUB
        gather_of(r, s).wait()

        if s < NSUB - 1:
          @pl.when(r >= 1)
          def _():
            write_of(r, ns).wait()
          gather_of(r, s + 1).start()
        else:
          write_of(r, ns).wait()
          @pl.when(r <= ROWS_PER_W - 2)
          def _():
            gather_of(r + 1, 0).start()

        def add_pos(i, _):
          t = T0[s] + i
          p = jnp.minimum(t, N_TOKENS - 1) if s == NSUB - 1 else t
          for j in range(NVEC):
            sl = pl.ds(j * L, L)
            plsc.addupdate(bufs[s].at[i, sl], pos_v[p, sl])
          return 0

        lax.fori_loop(0, GN[s], add_pos, 0)
        write_of(r, s).start()
      return 0

    lax.fori_loop(0, ROWS_PER_W, per_batch_row, 0)
    for s in range(1, NSUB):
      write_of(ROWS_PER_W - 1, s).wait()

  return emb_kernel


_EMB_KERNEL = _make_kernel()


def kernel(tokens, token_embedding, position_embedding):
  tok_pad = jnp.pad(tokens.astype(jnp.int32), ((0, 0), (0, TPAD - N_TOKENS)))
  return _EMB_KERNEL(tok_pad, token_embedding, position_embedding)


# speedup vs baseline: 1.6800x; 1.1219x over previous
import functools

import jax
import jax.numpy as jnp
from jax import lax
from jax.experimental import pallas as pl
from jax.experimental.pallas import tpu as pltpu
from jax.experimental.pallas import tpu_sc as plsc

N_VOCAB = 49408
N_EMBD = 768
N_TOKENS = 77
BATCH = 256

NC = 2
NS = 16
L = 16
NW = NC * NS
ROWS_PER_W = BATCH // NW
TPAD = 80
NSUB = 4
T0 = (0, 24, 48, 72)
GN = (24, 24, 24, 8)
WN = (24, 24, 24, N_TOKENS - 72)
NVEC = N_EMBD // L


def _make_kernel():
  mesh = plsc.VectorSubcoreMesh(core_axis_name="c", subcore_axis_name="s")

  @functools.partial(
      pl.kernel,
      mesh=mesh,
      out_type=jax.ShapeDtypeStruct((BATCH, N_TOKENS, N_EMBD), jnp.float32),
      scratch_types=[
          pltpu.VMEM((ROWS_PER_W, TPAD), jnp.int32),
          pltpu.VMEM((N_TOKENS, N_EMBD), jnp.float32),
          pltpu.VMEM((GN[0], N_EMBD), jnp.float32),
          pltpu.VMEM((GN[1], N_EMBD), jnp.float32),
          pltpu.VMEM((GN[2], N_EMBD), jnp.float32),
          pltpu.VMEM((GN[3], N_EMBD), jnp.float32),
          pltpu.SemaphoreType.DMA,
          pltpu.SemaphoreType.DMA,
          pltpu.SemaphoreType.DMA,
          pltpu.SemaphoreType.DMA,
          pltpu.SemaphoreType.DMA,
          pltpu.SemaphoreType.DMA,
          pltpu.SemaphoreType.DMA,
          pltpu.SemaphoreType.DMA,
      ],
  )
  def emb_kernel(tok_hbm, table_hbm, pos_hbm, out_hbm, idx8_v, pos_v,
                 buf0, buf1, buf2, buf3, gsem0, gsem1, gsem2, gsem3,
                 wsem0, wsem1, wsem2, wsem3):
    bufs = (buf0, buf1, buf2, buf3)
    gsems = (gsem0, gsem1, gsem2, gsem3)
    wsems = (wsem0, wsem1, wsem2, wsem3)
    wid = lax.axis_index("s") * NC + lax.axis_index("c")
    row_base = wid * ROWS_PER_W

    def gather_of(r, s):
      return pltpu.make_async_copy(
          table_hbm.at[idx8_v.at[r, pl.ds(T0[s], GN[s])]],
          bufs[s].at[pl.ds(0, GN[s])], gsems[s])

    def write_of(r, s):
      return pltpu.make_async_copy(
          bufs[s].at[pl.ds(0, WN[s])],
          out_hbm.at[row_base + r, pl.ds(T0[s], WN[s])], wsems[s])

    pltpu.sync_copy(tok_hbm.at[pl.ds(row_base, ROWS_PER_W)], idx8_v)
    gather_of(0, 0).start()
    gather_of(0, 1).start()
    pltpu.sync_copy(pos_hbm, pos_v)

    def per_batch_row(r, _):
      for s in range(NSUB):
        ns = (s + 2) % NSUB
        gather_of(r, s).wait()

        if s < 2:
          @pl.when(r >= 1)
          def _():
            write_of(r, ns).wait()
          gather_of(r, s + 2).start()
        else:
          write_of(r, ns).wait()
          @pl.when(r <= ROWS_PER_W - 2)
          def _():
            gather_of(r + 1, s - 2).start()

        def add_pos(i, _):
          t = T0[s] + i
          p = jnp.minimum(t, N_TOKENS - 1) if s == NSUB - 1 else t
          for j in range(NVEC):
            sl = pl.ds(j * L, L)
            plsc.addupdate(bufs[s].at[i, sl], pos_v[p, sl])
          return 0

        lax.fori_loop(0, GN[s], add_pos, 0)
        write_of(r, s).start()
      return 0

    lax.fori_loop(0, ROWS_PER_W, per_batch_row, 0)
    for s in range(2, NSUB):
      write_of(ROWS_PER_W - 1, s).wait()

  return emb_kernel


_EMB_KERNEL = _make_kernel()


def kernel(tokens, token_embedding, position_embedding):
  tok_pad = jnp.pad(tokens.astype(jnp.int32), ((0, 0), (0, TPAD - N_TOKENS)))
  return _EMB_KERNEL(tok_pad, token_embedding, position_embedding)


# speedup vs baseline: 4.5164x; 2.6884x over previous
import functools

import jax
import jax.numpy as jnp
from jax import lax
from jax.experimental import pallas as pl
from jax.experimental.pallas import tpu as pltpu
from jax.experimental.pallas import tpu_sc as plsc

N_VOCAB = 49408
N_EMBD = 768
N_TOKENS = 77
BATCH = 256

NC = 2
NS = 16
L = 16
NW = NC * NS
CB = 32
QPT = BATCH // CB
NCHUNK = N_TOKENS * QPT
KMAX = 20
NSTEP = 21
NVEC = N_EMBD // L
JB = 16


def _make_kernel():
  mesh = plsc.VectorSubcoreMesh(core_axis_name="c", subcore_axis_name="s")

  @functools.partial(
      pl.kernel,
      mesh=mesh,
      out_type=jax.ShapeDtypeStruct((N_TOKENS, BATCH, N_EMBD), jnp.float32),
      scratch_types=[
          pltpu.VMEM((KMAX, CB), jnp.int32),
          pltpu.VMEM((KMAX, N_EMBD), jnp.float32),
          pltpu.VMEM((CB, N_EMBD), jnp.float32),
          pltpu.VMEM((CB, N_EMBD), jnp.float32),
          pltpu.VMEM((CB, N_EMBD), jnp.float32),
          pltpu.SemaphoreType.DMA,
          pltpu.SemaphoreType.DMA,
          pltpu.SemaphoreType.DMA,
          pltpu.SemaphoreType.DMA,
          pltpu.SemaphoreType.DMA,
          pltpu.SemaphoreType.DMA,
      ],
  )
  def emb_kernel(idx_hbm, table_hbm, pos_hbm, out_hbm, idx_v, pos_v,
                 buf0, buf1, buf2, gsem0, gsem1, gsem2, wsem0, wsem1, wsem2):
    bufs = (buf0, buf1, buf2)
    gsems = (gsem0, gsem1, gsem2)
    wsems = (wsem0, wsem1, wsem2)
    wid = lax.axis_index("s") * NC + lax.axis_index("c")

    def gather_of(k, b):
      return pltpu.make_async_copy(
          table_hbm.at[idx_v.at[k]], bufs[b], gsems[b])

    def write_of(k, b):
      g = wid + NW * k
      t = g // QPT
      qoff = pl.multiple_of((g % QPT) * CB, CB)
      return pltpu.make_async_copy(
          bufs[b], out_hbm.at[t, pl.ds(qoff, CB)], wsems[b])

    def valid(k):
      return wid + NW * k < NCHUNK

    pltpu.sync_copy(idx_hbm.at[wid], idx_v)
    gather_of(0, 0).start()
    gather_of(1, 1).start()
    pltpu.sync_copy(pos_hbm.at[wid], pos_v)

    def per_round(r, _):
      for s in range(3):
        k = r * 3 + s

        @pl.when(valid(k))
        def _():
          gather_of(k, s).wait()

        pred_w = valid(k - 1) if s > 0 else jnp.logical_and(r >= 1,
                                                            valid(k - 1))
        @pl.when(pred_w)
        def _():
          write_of(k - 1, (s + 2) % 3).wait()

        @pl.when(valid(k + 2))
        def _():
          gather_of(k + 2, (s + 2) % 3).start()

        @pl.when(valid(k))
        def _():
          for jb in range(NVEC // JB):
            pregs = [pos_v[k, pl.ds((jb * JB + j) * L, L)] for j in range(JB)]

            def add_block(i, c):
              for j in range(JB):
                plsc.addupdate(bufs[s].at[i, pl.ds((jb * JB + j) * L, L)],
                               pregs[j])
              return c

            lax.fori_loop(0, CB, add_block, 0)
          write_of(k, s).start()
      return 0

    lax.fori_loop(0, NSTEP // 3, per_round, 0)

  return emb_kernel


_EMB_KERNEL = _make_kernel()


def kernel(tokens, token_embedding, position_embedding):
  tok_t = tokens.astype(jnp.int32).T
  idx_all = tok_t.reshape(NCHUNK, CB)
  idx_all = jnp.pad(idx_all, ((0, KMAX * NW - NCHUNK), (0, 0)))
  idx_all = idx_all.reshape(KMAX, NW, CB).transpose(1, 0, 2)
  pos_all = jnp.repeat(position_embedding, QPT, axis=0)
  pos_all = jnp.pad(pos_all, ((0, KMAX * NW - NCHUNK), (0, 0)))
  pos_all = pos_all.reshape(KMAX, NW, N_EMBD).transpose(1, 0, 2)
  out_t = _EMB_KERNEL(idx_all, token_embedding, pos_all)
  return jnp.transpose(out_t, (1, 0, 2))
